# baseline (speedup 1.0000x reference)
import jax
import jax.numpy as jnp
from jax import lax
from jax.experimental import pallas as pl
from jax.experimental.pallas import tpu as pltpu
from jax.experimental.pallas import tpu_sc as plsc

NC = 2
NS = 16
NW = NC * NS
B = 4096
D = 64
PLEN = 20
PPAD = 32
BPW = B // NW
VP = 500000


def _sc_gather_body(idx_hbm, w0p_hbm, out_hbm, idx_v, rows_v, sem):
    wid = lax.axis_index("s") * NC + lax.axis_index("c")
    base = wid * BPW
    pltpu.sync_copy(idx_hbm.at[pl.ds(base, BPW)], idx_v)
    pltpu.async_copy(w0p_hbm.at[idx_v], rows_v, sem).wait()
    pltpu.sync_copy(rows_v, out_hbm.at[pl.ds(base, BPW)])


def _sc_gather(pair_idx, W0p):
    mesh = plsc.VectorSubcoreMesh(core_axis_name="c", subcore_axis_name="s",
                                  num_cores=NC, num_subcores=NS)
    return pl.kernel(
        _sc_gather_body,
        out_type=jax.ShapeDtypeStruct((B, 2 * D), jnp.float32),
        mesh=mesh,
        scratch_types=[
            pltpu.VMEM((BPW,), jnp.int32),
            pltpu.VMEM((BPW, 2 * D), jnp.float32),
            pltpu.SemaphoreType.DMA,
        ],
    )(pair_idx, W0p)


def _tc_body(p0i_ref, c2_ref, par_ref, labels_ref, w1_ref, out_ref,
             p0b, psem):
    for j in range(PLEN):
        pltpu.make_async_copy(w1_ref.at[pl.ds(p0i_ref[j], 1)],
                              p0b.at[pl.ds(j, 1)], psem).start()
    pltpu.make_async_copy(w1_ref.at[pl.ds(0, PLEN)],
                          p0b.at[pl.ds(0, PLEN)], psem).wait()

    c2 = c2_ref[...]
    par = par_ref[...]
    c = jnp.where(par > 0.5, c2[:, D:], c2[:, :D])
    p = p0b[...]
    scores = lax.dot_general(c, p, (((1,), (1,)), ((), ())),
                             preferred_element_type=jnp.float32)
    s = scores[:, :PLEN]
    lab = labels_ref[...]
    z = jnp.log(1.0 / (1.0 + jnp.exp(-s)))
    log_z = jnp.maximum(jnp.log(z), -100.0)
    log_1mz = jnp.maximum(jnp.log(1.0 - z), -100.0)
    out_ref[0, 0] = -jnp.sum(lab * log_z + (1.0 - lab) * log_1mz)


def _tc_loss(paths0, c2, parity, labels, W1):
    out = pl.pallas_call(
        _tc_body,
        out_shape=jax.ShapeDtypeStruct((1, 1), jnp.float32),
        in_specs=[
            pl.BlockSpec(memory_space=pltpu.SMEM),
            pl.BlockSpec(memory_space=pltpu.VMEM),
            pl.BlockSpec(memory_space=pltpu.VMEM),
            pl.BlockSpec(memory_space=pltpu.VMEM),
            pl.BlockSpec(memory_space=pl.ANY),
        ],
        out_specs=pl.BlockSpec(memory_space=pltpu.SMEM),
        scratch_shapes=[
            pltpu.VMEM((PPAD, D), jnp.float32),
            pltpu.SemaphoreType.DMA,
        ],
    )(paths0, c2, parity, labels, W1)
    return out[0, 0]


def kernel(c_words, paths, labels, W0, W1):
    c_words = jnp.squeeze(c_words).astype(jnp.int32)
    paths0 = jnp.squeeze(paths)[0].astype(jnp.int32)
    labels = jnp.squeeze(labels)
    W0p = jnp.pad(jnp.reshape(W0, (-1,)), (0, 2 * D))[: VP * 2 * D]
    W0p = jnp.reshape(W0p, (VP, 2 * D))
    pair_idx = c_words >> 1
    parity = (c_words & 1).astype(jnp.float32).reshape(B, 1)
    c2 = _sc_gather(pair_idx, W0p)
    return _tc_loss(paths0, c2, parity, labels, W1)
